# Initial kernel scaffold; baseline (speedup 1.0000x reference)
import jax
import jax.numpy as jnp
from jax.experimental import pallas as pl


def kernel(score, update_slot, startProb, endProb, slotValueProb, cata_target, cate_mask, noncate_start, noncate_end, noncate_mask):
    raise NotImplementedError("write your pallas kernel here")


# speedup vs baseline: 1.4753x; 1.4753x over previous
import functools

import jax
import jax.numpy as jnp
from jax import lax
from jax.experimental import pallas as pl
from jax.experimental.pallas import tpu as pltpu
from jax.experimental.pallas import tpu_sc as plsc

B = 64
S = 30
H = 20
MAXV = 256
PAD = 512
N = B * S

PER_W = 128
ACT_W = N // PER_W
NCH = PER_W // 16

_LN2 = 0.6931471805599453
_SQRT2 = 1.4142135381698608


def _vlog(x):
    xi = lax.bitcast_convert_type(x, jnp.int32)
    e = lax.shift_right_arithmetic(xi, 23) - 127
    m = lax.bitcast_convert_type((xi & 0x007FFFFF) | 0x3F800000, jnp.float32)
    big = m > _SQRT2
    e = e + big.astype(jnp.int32)
    m = jnp.where(big, m * 0.5, m)
    t = (m - 1.0) / (m + 1.0)
    z = t * t
    p = 2.0 + z * (0.66666668653488159 + z * (0.40000000596046448
        + z * (0.28571429848670959 + z * 0.22222222222222222)))
    return e.astype(jnp.float32) * _LN2 + t * p


@functools.partial(
    pl.kernel,
    mesh=plsc.VectorSubcoreMesh(core_axis_name="c", subcore_axis_name="s",
                                num_cores=1),
    out_type=jax.ShapeDtypeStruct((16,), jnp.float32),
    compiler_params=pltpu.CompilerParams(needs_layout_passes=False),
    scratch_types=[
        pltpu.VMEM((PER_W * H,), jnp.float32),
        pltpu.VMEM((PER_W,), jnp.int32),
        pltpu.VMEM((PER_W,), jnp.int32),
        pltpu.VMEM((PER_W,), jnp.int32),
        pltpu.VMEM((PER_W,), jnp.int32),
        pltpu.VMEM((PER_W,), jnp.int32),
        pltpu.VMEM((PER_W,), jnp.int32),
        pltpu.VMEM((PER_W,), jnp.int32),
        pltpu.VMEM((PER_W,), jnp.int32),
        pltpu.VMEM((PER_W,), jnp.int32),
        pltpu.VMEM((PER_W,), jnp.int32),
        pltpu.VMEM((PER_W,), jnp.float32),
        pltpu.VMEM((PER_W,), jnp.float32),
        pltpu.VMEM((PER_W,), jnp.float32),
        pltpu.VMEM((5 * 16,), jnp.float32),
        pltpu.VMEM((ACT_W * 5 * 16,), jnp.float32),
        pltpu.VMEM((16,), jnp.float32),
        pltpu.VMEM((16,), jnp.float32),
        pltpu.VMEM_SHARED((ACT_W * 5 * 16,), jnp.float32),
        pltpu.SemaphoreType.DMA,
    ],
)
def _sc_loss(score_hbm, upd_hbm, cm_hbm, ct_hbm, ns_hbm, ne_hbm, nm_hbm,
             svp_hbm, sp_hbm, ep_hbm, out_hbm,
             score_v, upd_v, midx_v, cm_v, ct_v, ns_v, ne_v, nm_v,
             svpi_v, spi_v, epi_v, svpv_v, spv_v, epv_v,
             acc_v, all_v, tmp_v, outv_v, shared, sem):
    wid = lax.axis_index("s")

    @pl.when(wid < ACT_W)
    def _work():
        base = wid * PER_W
        pltpu.sync_copy(score_hbm.at[pl.ds(base * H, PER_W * H)], score_v)
        pltpu.sync_copy(upd_hbm.at[pl.ds(base, PER_W)], upd_v)
        lanes = lax.broadcasted_iota(jnp.int32, (16,), 0)
        for c in range(NCH):
            nloc = c * 16 + lanes
            n = base + nloc
            sbase = nloc * H
            best = plsc.load_gather(score_v, [sbase])
            bidx = jnp.zeros((16,), jnp.int32)
            for h in range(1, H):
                v = plsc.load_gather(score_v, [sbase + h])
                better = v > best
                best = jnp.where(better, v, best)
                bidx = jnp.where(better, h, bidx)
            b = n // S
            s = n - b * S
            midx_v[pl.ds(c * 16, 16)] = b * (H * S) + bidx * S + s
        cps = [pltpu.async_copy(cm_hbm.at[midx_v], cm_v, sem),
               pltpu.async_copy(ct_hbm.at[midx_v], ct_v, sem),
               pltpu.async_copy(ns_hbm.at[midx_v], ns_v, sem),
               pltpu.async_copy(ne_hbm.at[midx_v], ne_v, sem),
               pltpu.async_copy(nm_hbm.at[midx_v], nm_v, sem)]
        for cp in cps:
            cp.wait()
        for c in range(NCH):
            sl = pl.ds(c * 16, 16)
            n = base + c * 16 + lanes
            b = n // S
            s = n - b * S
            svpi_v[sl] = n * (S * MAXV) + s * MAXV + ct_v[sl]
            spi_v[sl] = n * (S * PAD) + s * PAD + ns_v[sl]
            epi_v[sl] = n * (S * PAD) + s * PAD + ne_v[sl]
        cps = [pltpu.async_copy(svp_hbm.at[svpi_v], svpv_v, sem),
               pltpu.async_copy(sp_hbm.at[spi_v], spv_v, sem),
               pltpu.async_copy(ep_hbm.at[epi_v], epv_v, sem)]
        for cp in cps:
            cp.wait()
        zero = jnp.zeros((16,), jnp.float32)
        cls_sum, cls_cnt = zero, zero
        s_sum, e_sum, ext_cnt = zero, zero, zero
        for c in range(NCH):
            sl = pl.ds(c * 16, 16)
            ok = upd_v[sl] == 1
            wcls = (cm_v[sl] == 1) & ok
            wext = (nm_v[sl] == 1) & ok
            cls_sum = cls_sum + jnp.where(wcls, -_vlog(svpv_v[sl] + 1e-5), 0.0)
            cls_cnt = cls_cnt + jnp.where(wcls, 1.0, 0.0)
            s_sum = s_sum + jnp.where(wext, -_vlog(spv_v[sl] + 1e-5), 0.0)
            e_sum = e_sum + jnp.where(wext, -_vlog(epv_v[sl] + 1e-5), 0.0)
            ext_cnt = ext_cnt + jnp.where(wext, 1.0, 0.0)
        acc_v[pl.ds(0, 16)] = cls_sum
        acc_v[pl.ds(16, 16)] = cls_cnt
        acc_v[pl.ds(32, 16)] = s_sum
        acc_v[pl.ds(48, 16)] = e_sum
        acc_v[pl.ds(64, 16)] = ext_cnt
        pltpu.sync_copy(acc_v, shared.at[pl.ds(wid * 80, 80)])

    plsc.subcore_barrier()

    @pl.when(wid == 0)
    def _reduce():
        pltpu.sync_copy(shared, all_v)
        tot = [jnp.zeros((16,), jnp.float32) for _ in range(5)]
        for w in range(ACT_W):
            for r in range(5):
                tot[r] = tot[r] + all_v[pl.ds(w * 80 + r * 16, 16)]
        full15 = jnp.full((16,), 15, jnp.int32)

        def lanesum(v):
            tmp_v[...] = jnp.cumsum(v)
            return plsc.load_gather(tmp_v, [full15])

        cls_sum = lanesum(tot[0])
        cls_cnt = lanesum(tot[1])
        s_sum = lanesum(tot[2])
        e_sum = lanesum(tot[3])
        ext_cnt = lanesum(tot[4])
        cls = jnp.where(cls_cnt > 0, cls_sum / jnp.maximum(cls_cnt, 1.0), 0.0)
        stl = jnp.where(ext_cnt > 0, s_sum / jnp.maximum(ext_cnt, 1.0), 0.0)
        enl = jnp.where(ext_cnt > 0, e_sum / jnp.maximum(ext_cnt, 1.0), 0.0)
        outv_v[...] = cls + stl + enl
        pltpu.sync_copy(outv_v, out_hbm)


def kernel(score, update_slot, startProb, endProb, slotValueProb,
           cata_target, cate_mask, noncate_start, noncate_end, noncate_mask):
    out16 = _sc_loss(
        score.reshape(-1).astype(jnp.float32),
        update_slot.reshape(-1).astype(jnp.int32),
        cate_mask.reshape(-1).astype(jnp.int32),
        cata_target.reshape(-1).astype(jnp.int32),
        noncate_start.reshape(-1).astype(jnp.int32),
        noncate_end.reshape(-1).astype(jnp.int32),
        noncate_mask.reshape(-1).astype(jnp.int32),
        slotValueProb.reshape(-1),
        startProb.reshape(-1),
        endProb.reshape(-1),
    )
    return out16[0]


# speedup vs baseline: 2.3156x; 1.5695x over previous
import functools

import jax
import jax.numpy as jnp
from jax import lax
from jax.experimental import pallas as pl
from jax.experimental.pallas import tpu as pltpu
from jax.experimental.pallas import tpu_sc as plsc

B = 64
S = 30
H = 20
MAXV = 256
PAD = 512
N = B * S

PER_W = 128
ACT_W = N // PER_W
NCH = PER_W // 16
WIN = 16

_LN2 = 0.6931471805599453
_SQRT2 = 1.4142135381698608


def _vlog(x):
    xi = lax.bitcast_convert_type(x, jnp.int32)
    e = lax.shift_right_arithmetic(xi, 23) - 127
    m = lax.bitcast_convert_type((xi & 0x007FFFFF) | 0x3F800000, jnp.float32)
    big = m > _SQRT2
    e = e + big.astype(jnp.int32)
    m = jnp.where(big, m * 0.5, m)
    t = (m - 1.0) / (m + 1.0)
    z = t * t
    p = 2.0 + z * (0.66666668653488159 + z * (0.40000000596046448
        + z * (0.28571429848670959 + z * 0.22222222222222222)))
    return e.astype(jnp.float32) * _LN2 + t * p


@functools.partial(
    pl.kernel,
    mesh=plsc.VectorSubcoreMesh(core_axis_name="c", subcore_axis_name="s",
                                num_cores=1),
    out_type=jax.ShapeDtypeStruct((16,), jnp.float32),
    compiler_params=pltpu.CompilerParams(needs_layout_passes=False,
                                         disable_bounds_checks=True),
    scratch_types=[
        pltpu.VMEM((PER_W * H,), jnp.float32),
        pltpu.VMEM((PER_W,), jnp.int32),
        pltpu.VMEM((PER_W,), jnp.int32),
        pltpu.VMEM((PER_W,), jnp.int32),
        pltpu.VMEM((PER_W,), jnp.int32),
        pltpu.VMEM((PER_W,), jnp.int32),
        pltpu.VMEM((PER_W,), jnp.int32),
        pltpu.VMEM((PER_W,), jnp.int32),
        pltpu.VMEM((PER_W,), jnp.int32),
        pltpu.VMEM((16, 8, 128), jnp.float32),
        pltpu.VMEM((16, 8, 128), jnp.float32),
        pltpu.VMEM((16, 8, 128), jnp.float32),
        pltpu.VMEM((5 * 16,), jnp.float32),
        pltpu.VMEM((ACT_W * 5 * 16,), jnp.float32),
        pltpu.VMEM((16,), jnp.float32),
        pltpu.VMEM((16,), jnp.float32),
        pltpu.VMEM_SHARED((ACT_W * 5 * 16,), jnp.float32),
        pltpu.SemaphoreType.DMA,
    ],
)
def _sc_loss(score_hbm, upd_hbm, cm_hbm, ct_hbm, ns_hbm, ne_hbm, nm_hbm,
             svp_hbm, sp_hbm, ep_hbm, out_hbm,
             score_v, upd_v, midx_v, s_v, cm_v, ct_v, ns_v, ne_v, nm_v,
             svpw_v, spw_v, epw_v,
             acc_v, all_v, tmp_v, outv_v, shared, sem):
    wid = lax.axis_index("s")

    @pl.when(wid < ACT_W)
    def _work():
        base = wid * PER_W
        pltpu.sync_copy(score_hbm.at[pl.ds(base * H, PER_W * H)], score_v)
        pltpu.sync_copy(upd_hbm.at[pl.ds(base, PER_W)], upd_v)
        lanes = lax.broadcasted_iota(jnp.int32, (16,), 0)
        for c in range(NCH):
            nloc = c * 16 + lanes
            n = base + nloc
            sbase = nloc * H
            best = plsc.load_gather(score_v, [sbase])
            bidx = jnp.zeros((16,), jnp.int32)
            for h in range(1, H):
                v = plsc.load_gather(score_v, [sbase + h])
                better = v > best
                best = jnp.where(better, v, best)
                bidx = jnp.where(better, h, bidx)
            b = n // S
            s = n - b * S
            s_v[pl.ds(c * 16, 16)] = s
            midx_v[pl.ds(c * 16, 16)] = b * (H * S) + bidx * S + s
        cps = [pltpu.async_copy(cm_hbm.at[midx_v], cm_v, sem),
               pltpu.async_copy(ct_hbm.at[midx_v], ct_v, sem),
               pltpu.async_copy(ns_hbm.at[midx_v], ns_v, sem),
               pltpu.async_copy(ne_hbm.at[midx_v], ne_v, sem),
               pltpu.async_copy(nm_hbm.at[midx_v], nm_v, sem)]
        for cp in cps:
            cp.wait()
        zero = jnp.zeros((16,), jnp.float32)
        cls_sum, cls_cnt = zero, zero
        s_sum, e_sum, ext_cnt = zero, zero, zero
        for c in range(NCH):
            sl = pl.ds(c * 16, 16)
            s_vec = s_v[sl]
            sb = s_vec & -8
            a1 = ct_v[sl] & -128
            a2 = ns_v[sl] & -128
            a3 = ne_v[sl] & -128
            for l in range(16):
                n = base + c * 16 + l
                sbl = pl.multiple_of(sb[l], 8)
                pltpu.async_copy(
                    svp_hbm.at[n, pl.ds(sbl, 8),
                               pl.ds(pl.multiple_of(a1[l], 128), 128)],
                    svpw_v.at[l], sem)
                pltpu.async_copy(
                    sp_hbm.at[n, pl.ds(sbl, 8),
                              pl.ds(pl.multiple_of(a2[l], 128), 128)],
                    spw_v.at[l], sem)
                pltpu.async_copy(
                    ep_hbm.at[n, pl.ds(sbl, 8),
                              pl.ds(pl.multiple_of(a3[l], 128), 128)],
                    epw_v.at[l], sem)
            for hbm, buf in ((svp_hbm, svpw_v), (sp_hbm, spw_v),
                             (ep_hbm, epw_v)):
                pltpu.make_async_copy(
                    hbm.at[pl.ds(0, 16), pl.ds(0, 8), pl.ds(0, 128)],
                    buf, sem).wait()
            srow = s_vec & 7
            sv = plsc.load_gather(svpw_v, [lanes, srow, ct_v[sl] & 127])
            st = plsc.load_gather(spw_v, [lanes, srow, ns_v[sl] & 127])
            en = plsc.load_gather(epw_v, [lanes, srow, ne_v[sl] & 127])
            ok = upd_v[sl] == 1
            wcls = (cm_v[sl] == 1) & ok
            wext = (nm_v[sl] == 1) & ok
            cls_sum = cls_sum + jnp.where(wcls, -_vlog(sv + 1e-5), 0.0)
            cls_cnt = cls_cnt + jnp.where(wcls, 1.0, 0.0)
            s_sum = s_sum + jnp.where(wext, -_vlog(st + 1e-5), 0.0)
            e_sum = e_sum + jnp.where(wext, -_vlog(en + 1e-5), 0.0)
            ext_cnt = ext_cnt + jnp.where(wext, 1.0, 0.0)
        acc_v[pl.ds(0, 16)] = cls_sum
        acc_v[pl.ds(16, 16)] = cls_cnt
        acc_v[pl.ds(32, 16)] = s_sum
        acc_v[pl.ds(48, 16)] = e_sum
        acc_v[pl.ds(64, 16)] = ext_cnt
        pltpu.sync_copy(acc_v, shared.at[pl.ds(wid * 80, 80)])

    plsc.subcore_barrier()

    @pl.when(wid == 0)
    def _reduce():
        pltpu.sync_copy(shared, all_v)
        tot = [jnp.zeros((16,), jnp.float32) for _ in range(5)]
        for w in range(ACT_W):
            for r in range(5):
                tot[r] = tot[r] + all_v[pl.ds(w * 80 + r * 16, 16)]
        full15 = jnp.full((16,), 15, jnp.int32)

        def lanesum(v):
            tmp_v[...] = jnp.cumsum(v)
            return plsc.load_gather(tmp_v, [full15])

        cls_sum = lanesum(tot[0])
        cls_cnt = lanesum(tot[1])
        s_sum = lanesum(tot[2])
        e_sum = lanesum(tot[3])
        ext_cnt = lanesum(tot[4])
        cls = jnp.where(cls_cnt > 0, cls_sum / jnp.maximum(cls_cnt, 1.0), 0.0)
        stl = jnp.where(ext_cnt > 0, s_sum / jnp.maximum(ext_cnt, 1.0), 0.0)
        enl = jnp.where(ext_cnt > 0, e_sum / jnp.maximum(ext_cnt, 1.0), 0.0)
        outv_v[...] = cls + stl + enl
        pltpu.sync_copy(outv_v, out_hbm)


def kernel(score, update_slot, startProb, endProb, slotValueProb,
           cata_target, cate_mask, noncate_start, noncate_end, noncate_mask):
    out16 = _sc_loss(
        score.reshape(-1).astype(jnp.float32),
        update_slot.reshape(-1).astype(jnp.int32),
        cate_mask.reshape(-1).astype(jnp.int32),
        cata_target.reshape(-1).astype(jnp.int32),
        noncate_start.reshape(-1).astype(jnp.int32),
        noncate_end.reshape(-1).astype(jnp.int32),
        noncate_mask.reshape(-1).astype(jnp.int32),
        slotValueProb,
        startProb,
        endProb,
    )
    return out16[0]


# speedup vs baseline: 11.9750x; 5.1714x over previous
import functools

import jax
import jax.numpy as jnp
from jax import lax
from jax.experimental import pallas as pl
from jax.experimental.pallas import tpu as pltpu
from jax.experimental.pallas import tpu_sc as plsc

B = 64
S = 30
H = 20
MAXV = 256
PAD = 512
N = B * S

PER_W = 128
ACT_W = N // PER_W
NCH = PER_W // 16
WIN = 16

_LN2 = 0.6931471805599453
_SQRT2 = 1.4142135381698608


def _vlog(x):
    xi = lax.bitcast_convert_type(x, jnp.int32)
    e = lax.shift_right_arithmetic(xi, 23) - 127
    m = lax.bitcast_convert_type((xi & 0x007FFFFF) | 0x3F800000, jnp.float32)
    big = m > _SQRT2
    e = e + big.astype(jnp.int32)
    m = jnp.where(big, m * 0.5, m)
    t = (m - 1.0) / (m + 1.0)
    z = t * t
    p = 2.0 + z * (0.66666668653488159 + z * (0.40000000596046448
        + z * (0.28571429848670959 + z * 0.22222222222222222)))
    return e.astype(jnp.float32) * _LN2 + t * p


@functools.partial(
    pl.kernel,
    mesh=plsc.VectorSubcoreMesh(core_axis_name="c", subcore_axis_name="s",
                                num_cores=1),
    out_type=jax.ShapeDtypeStruct((16,), jnp.float32),
    compiler_params=pltpu.CompilerParams(needs_layout_passes=False,
                                         disable_bounds_checks=True),
    scratch_types=[
        pltpu.VMEM((PER_W * H,), jnp.float32),
        pltpu.VMEM((PER_W,), jnp.int32),
        pltpu.VMEM((PER_W,), jnp.int32),
        pltpu.VMEM((PER_W,), jnp.int32),
        pltpu.VMEM((PER_W,), jnp.int32),
        pltpu.VMEM((PER_W,), jnp.int32),
        pltpu.VMEM((PER_W,), jnp.int32),
        pltpu.VMEM((PER_W,), jnp.int32),
        pltpu.VMEM((PER_W,), jnp.int32),
        pltpu.VMEM((16, 8, 128), jnp.float32),
        pltpu.VMEM((16, 8, 128), jnp.float32),
        pltpu.VMEM((16, 8, 128), jnp.float32),
        pltpu.VMEM((5 * 16,), jnp.float32),
        pltpu.VMEM((ACT_W * 5 * 16,), jnp.float32),
        pltpu.VMEM((16,), jnp.float32),
        pltpu.VMEM((16,), jnp.float32),
        pltpu.VMEM_SHARED((ACT_W * 5 * 16,), jnp.float32),
        pltpu.SemaphoreType.DMA,
    ],
)
def _sc_loss(score_hbm, upd_hbm, cm_hbm, ct_hbm, ns_hbm, ne_hbm, nm_hbm,
             svp_hbm, sp_hbm, ep_hbm, out_hbm,
             score_v, upd_v, midx_v, s_v, cm_v, ct_v, ns_v, ne_v, nm_v,
             svpw_v, spw_v, epw_v,
             acc_v, all_v, tmp_v, outv_v, shared, sem):
    wid = lax.axis_index("s")

    @pl.when(wid < ACT_W)
    def _work():
        base = wid * PER_W
        pltpu.sync_copy(score_hbm.at[pl.ds(base * H, PER_W * H)], score_v)
        pltpu.sync_copy(upd_hbm.at[pl.ds(base, PER_W)], upd_v)
        lanes = lax.broadcasted_iota(jnp.int32, (16,), 0)
        for c in range(NCH):
            nloc = c * 16 + lanes
            n = base + nloc
            sbase = nloc * H
            best = plsc.load_gather(score_v, [sbase])
            bidx = jnp.zeros((16,), jnp.int32)
            for h in range(1, H):
                v = plsc.load_gather(score_v, [sbase + h])
                better = v > best
                best = jnp.where(better, v, best)
                bidx = jnp.where(better, h, bidx)
            b = n // S
            s = n - b * S
            s_v[pl.ds(c * 16, 16)] = s
            midx_v[pl.ds(c * 16, 16)] = b * (H * S) + bidx * S + s
        cps = [pltpu.async_copy(cm_hbm.at[midx_v], cm_v, sem),
               pltpu.async_copy(ct_hbm.at[midx_v], ct_v, sem),
               pltpu.async_copy(ns_hbm.at[midx_v], ns_v, sem),
               pltpu.async_copy(ne_hbm.at[midx_v], ne_v, sem),
               pltpu.async_copy(nm_hbm.at[midx_v], nm_v, sem)]
        for cp in cps:
            cp.wait()
        zero = jnp.zeros((16,), jnp.float32)
        cls_sum, cls_cnt = zero, zero
        s_sum, e_sum, ext_cnt = zero, zero, zero
        for c in range(NCH):
            sl = pl.ds(c * 16, 16)
            s_vec = s_v[sl]
            a1 = ct_v[sl] & -128
            a2 = ns_v[sl] & -128
            a3 = ne_v[sl] & -128
            for l in range(16):
                j = c * 16 + l
                nb = pl.multiple_of(base + (j & ~7), 8)
                s_l = s_vec[l]
                pltpu.async_copy(
                    svp_hbm.at[s_l, pl.ds(nb, 8),
                               pl.ds(pl.multiple_of(a1[l], 128), 128)],
                    svpw_v.at[l], sem)
                pltpu.async_copy(
                    sp_hbm.at[s_l, pl.ds(nb, 8),
                              pl.ds(pl.multiple_of(a2[l], 128), 128)],
                    spw_v.at[l], sem)
                pltpu.async_copy(
                    ep_hbm.at[s_l, pl.ds(nb, 8),
                              pl.ds(pl.multiple_of(a3[l], 128), 128)],
                    epw_v.at[l], sem)
            for hbm, buf in ((svp_hbm, svpw_v), (sp_hbm, spw_v),
                             (ep_hbm, epw_v)):
                pltpu.make_async_copy(
                    hbm.at[pl.ds(0, 16), pl.ds(0, 8), pl.ds(0, 128)],
                    buf, sem).wait()
            nrow = lanes & 7
            sv = plsc.load_gather(svpw_v, [lanes, nrow, ct_v[sl] & 127])
            st = plsc.load_gather(spw_v, [lanes, nrow, ns_v[sl] & 127])
            en = plsc.load_gather(epw_v, [lanes, nrow, ne_v[sl] & 127])
            ok = upd_v[sl] == 1
            wcls = (cm_v[sl] == 1) & ok
            wext = (nm_v[sl] == 1) & ok
            cls_sum = cls_sum + jnp.where(wcls, -_vlog(sv + 1e-5), 0.0)
            cls_cnt = cls_cnt + jnp.where(wcls, 1.0, 0.0)
            s_sum = s_sum + jnp.where(wext, -_vlog(st + 1e-5), 0.0)
            e_sum = e_sum + jnp.where(wext, -_vlog(en + 1e-5), 0.0)
            ext_cnt = ext_cnt + jnp.where(wext, 1.0, 0.0)
        acc_v[pl.ds(0, 16)] = cls_sum
        acc_v[pl.ds(16, 16)] = cls_cnt
        acc_v[pl.ds(32, 16)] = s_sum
        acc_v[pl.ds(48, 16)] = e_sum
        acc_v[pl.ds(64, 16)] = ext_cnt
        pltpu.sync_copy(acc_v, shared.at[pl.ds(wid * 80, 80)])

    plsc.subcore_barrier()

    @pl.when(wid == 0)
    def _reduce():
        pltpu.sync_copy(shared, all_v)
        tot = [jnp.zeros((16,), jnp.float32) for _ in range(5)]
        for w in range(ACT_W):
            for r in range(5):
                tot[r] = tot[r] + all_v[pl.ds(w * 80 + r * 16, 16)]
        full15 = jnp.full((16,), 15, jnp.int32)

        def lanesum(v):
            tmp_v[...] = jnp.cumsum(v)
            return plsc.load_gather(tmp_v, [full15])

        cls_sum = lanesum(tot[0])
        cls_cnt = lanesum(tot[1])
        s_sum = lanesum(tot[2])
        e_sum = lanesum(tot[3])
        ext_cnt = lanesum(tot[4])
        cls = jnp.where(cls_cnt > 0, cls_sum / jnp.maximum(cls_cnt, 1.0), 0.0)
        stl = jnp.where(ext_cnt > 0, s_sum / jnp.maximum(ext_cnt, 1.0), 0.0)
        enl = jnp.where(ext_cnt > 0, e_sum / jnp.maximum(ext_cnt, 1.0), 0.0)
        outv_v[...] = cls + stl + enl
        pltpu.sync_copy(outv_v, out_hbm)


def kernel(score, update_slot, startProb, endProb, slotValueProb,
           cata_target, cate_mask, noncate_start, noncate_end, noncate_mask):
    out16 = _sc_loss(
        score.reshape(-1).astype(jnp.float32),
        update_slot.reshape(-1).astype(jnp.int32),
        cate_mask.reshape(-1).astype(jnp.int32),
        cata_target.reshape(-1).astype(jnp.int32),
        noncate_start.reshape(-1).astype(jnp.int32),
        noncate_end.reshape(-1).astype(jnp.int32),
        noncate_mask.reshape(-1).astype(jnp.int32),
        slotValueProb.transpose(1, 0, 2),
        startProb.transpose(1, 0, 2),
        endProb.transpose(1, 0, 2),
    )
    return out16[0]


# speedup vs baseline: 16.1700x; 1.3503x over previous
import functools

import jax
import jax.numpy as jnp
from jax import lax
from jax.experimental import pallas as pl
from jax.experimental.pallas import tpu as pltpu
from jax.experimental.pallas import tpu_sc as plsc

B = 64
S = 30
H = 20
MAXV = 256
PAD = 512
N = B * S

PER_W = 128
ACT_W = N // PER_W
NCH = PER_W // 16
WIN = 16

_LN2 = 0.6931471805599453
_SQRT2 = 1.4142135381698608


def _vlog(x):
    xi = lax.bitcast_convert_type(x, jnp.int32)
    e = lax.shift_right_arithmetic(xi, 23) - 127
    m = lax.bitcast_convert_type((xi & 0x007FFFFF) | 0x3F800000, jnp.float32)
    big = m > _SQRT2
    e = e + big.astype(jnp.int32)
    m = jnp.where(big, m * 0.5, m)
    t = (m - 1.0) / (m + 1.0)
    z = t * t
    p = 2.0 + z * (0.66666668653488159 + z * (0.40000000596046448
        + z * (0.28571429848670959 + z * 0.22222222222222222)))
    return e.astype(jnp.float32) * _LN2 + t * p


@functools.partial(
    pl.kernel,
    mesh=plsc.VectorSubcoreMesh(core_axis_name="c", subcore_axis_name="s",
                                num_cores=1),
    out_type=jax.ShapeDtypeStruct((16,), jnp.float32),
    compiler_params=pltpu.CompilerParams(needs_layout_passes=False,
                                         disable_bounds_checks=True),
    scratch_types=[
        pltpu.VMEM((PER_W * H,), jnp.float32),
        pltpu.VMEM((PER_W,), jnp.int32),
        pltpu.VMEM((PER_W,), jnp.int32),
        pltpu.VMEM((PER_W,), jnp.int32),
        pltpu.VMEM((PER_W,), jnp.int32),
        pltpu.VMEM((PER_W,), jnp.int32),
        pltpu.VMEM((PER_W,), jnp.int32),
        pltpu.VMEM((PER_W,), jnp.int32),
        pltpu.VMEM((PER_W,), jnp.int32),
        pltpu.VMEM((PER_W,), jnp.int32),
        pltpu.VMEM((PER_W,), jnp.int32),
        pltpu.VMEM((PER_W,), jnp.int32),
        pltpu.VMEM((PER_W // 2, MAXV), jnp.float32),
        pltpu.VMEM((PER_W // 2, PAD), jnp.float32),
        pltpu.VMEM((PER_W // 2, PAD), jnp.float32),
        pltpu.VMEM((5 * 16,), jnp.float32),
        pltpu.VMEM((ACT_W * 5 * 16,), jnp.float32),
        pltpu.VMEM((16,), jnp.float32),
        pltpu.VMEM((16,), jnp.float32),
        pltpu.VMEM_SHARED((ACT_W * 5 * 16,), jnp.float32),
        pltpu.SemaphoreType.DMA,
    ],
)
def _sc_loss(score_hbm, upd_hbm, cm_hbm, ct_hbm, ns_hbm, ne_hbm, nm_hbm,
             svp_hbm, sp_hbm, ep_hbm, out_hbm,
             score_v, upd_v, midx_v, s_v, cm_v, ct_v, ns_v, ne_v, nm_v,
             svpi_v, spi_v, epi_v, svpv_v, spv_v, epv_v,
             acc_v, all_v, tmp_v, outv_v, shared, sem):
    wid = lax.axis_index("s")

    @pl.when(wid < ACT_W)
    def _work():
        base = wid * PER_W
        pltpu.sync_copy(score_hbm.at[pl.ds(base * H, PER_W * H)], score_v)
        pltpu.sync_copy(upd_hbm.at[pl.ds(base, PER_W)], upd_v)
        lanes = lax.broadcasted_iota(jnp.int32, (16,), 0)
        for c in range(NCH):
            nloc = c * 16 + lanes
            n = base + nloc
            sbase = nloc * H
            best = plsc.load_gather(score_v, [sbase])
            bidx = jnp.zeros((16,), jnp.int32)
            for h in range(1, H):
                v = plsc.load_gather(score_v, [sbase + h])
                better = v > best
                best = jnp.where(better, v, best)
                bidx = jnp.where(better, h, bidx)
            b = n // S
            s = n - b * S
            s_v[pl.ds(c * 16, 16)] = s
            midx_v[pl.ds(c * 16, 16)] = b * (H * S) + bidx * S + s
        cps = [pltpu.async_copy(cm_hbm.at[midx_v], cm_v, sem),
               pltpu.async_copy(ct_hbm.at[midx_v], ct_v, sem),
               pltpu.async_copy(ns_hbm.at[midx_v], ns_v, sem),
               pltpu.async_copy(ne_hbm.at[midx_v], ne_v, sem),
               pltpu.async_copy(nm_hbm.at[midx_v], nm_v, sem)]
        for cp in cps:
            cp.wait()
        for c in range(NCH):
            sl = pl.ds(c * 16, 16)
            n = base + c * 16 + lanes
            r = s_v[sl] * N + n
            svpi_v[sl] = r
            spi_v[sl] = r
            epi_v[sl] = r
        fl_svp = svp_hbm.reshape(S * N, MAXV)
        fl_sp = sp_hbm.reshape(S * N, PAD)
        fl_ep = ep_hbm.reshape(S * N, PAD)
        zero = jnp.zeros((16,), jnp.float32)
        cls_sum, cls_cnt = zero, zero
        s_sum, e_sum, ext_cnt = zero, zero, zero
        half_rows = PER_W // 2
        for half in range(2):
            hsl = pl.ds(half * half_rows, half_rows)
            cps = [pltpu.async_copy(fl_svp.at[svpi_v.at[hsl]], svpv_v, sem),
                   pltpu.async_copy(fl_sp.at[spi_v.at[hsl]], spv_v, sem),
                   pltpu.async_copy(fl_ep.at[epi_v.at[hsl]], epv_v, sem)]
            for cp in cps:
                cp.wait()
            for c in range(half_rows // 16):
                rows = c * 16 + lanes
                sl = pl.ds(half * half_rows + c * 16, 16)
                sv = plsc.load_gather(svpv_v, [rows, ct_v[sl]])
                st = plsc.load_gather(spv_v, [rows, ns_v[sl]])
                en = plsc.load_gather(epv_v, [rows, ne_v[sl]])
                ok = upd_v[sl] == 1
                wcls = (cm_v[sl] == 1) & ok
                wext = (nm_v[sl] == 1) & ok
                cls_sum = cls_sum + jnp.where(wcls, -_vlog(sv + 1e-5), 0.0)
                cls_cnt = cls_cnt + jnp.where(wcls, 1.0, 0.0)
                s_sum = s_sum + jnp.where(wext, -_vlog(st + 1e-5), 0.0)
                e_sum = e_sum + jnp.where(wext, -_vlog(en + 1e-5), 0.0)
                ext_cnt = ext_cnt + jnp.where(wext, 1.0, 0.0)
        acc_v[pl.ds(0, 16)] = cls_sum
        acc_v[pl.ds(16, 16)] = cls_cnt
        acc_v[pl.ds(32, 16)] = s_sum
        acc_v[pl.ds(48, 16)] = e_sum
        acc_v[pl.ds(64, 16)] = ext_cnt
        pltpu.sync_copy(acc_v, shared.at[pl.ds(wid * 80, 80)])

    plsc.subcore_barrier()

    @pl.when(wid == 0)
    def _reduce():
        pltpu.sync_copy(shared, all_v)
        tot = [jnp.zeros((16,), jnp.float32) for _ in range(5)]
        for w in range(ACT_W):
            for r in range(5):
                tot[r] = tot[r] + all_v[pl.ds(w * 80 + r * 16, 16)]
        full15 = jnp.full((16,), 15, jnp.int32)

        def lanesum(v):
            tmp_v[...] = jnp.cumsum(v)
            return plsc.load_gather(tmp_v, [full15])

        cls_sum = lanesum(tot[0])
        cls_cnt = lanesum(tot[1])
        s_sum = lanesum(tot[2])
        e_sum = lanesum(tot[3])
        ext_cnt = lanesum(tot[4])
        cls = jnp.where(cls_cnt > 0, cls_sum / jnp.maximum(cls_cnt, 1.0), 0.0)
        stl = jnp.where(ext_cnt > 0, s_sum / jnp.maximum(ext_cnt, 1.0), 0.0)
        enl = jnp.where(ext_cnt > 0, e_sum / jnp.maximum(ext_cnt, 1.0), 0.0)
        outv_v[...] = cls + stl + enl
        pltpu.sync_copy(outv_v, out_hbm)


def kernel(score, update_slot, startProb, endProb, slotValueProb,
           cata_target, cate_mask, noncate_start, noncate_end, noncate_mask):
    out16 = _sc_loss(
        score.reshape(-1).astype(jnp.float32),
        update_slot.reshape(-1).astype(jnp.int32),
        cate_mask.reshape(-1).astype(jnp.int32),
        cata_target.reshape(-1).astype(jnp.int32),
        noncate_start.reshape(-1).astype(jnp.int32),
        noncate_end.reshape(-1).astype(jnp.int32),
        noncate_mask.reshape(-1).astype(jnp.int32),
        slotValueProb.transpose(1, 0, 2),
        startProb.transpose(1, 0, 2),
        endProb.transpose(1, 0, 2),
    )
    return out16[0]


# speedup vs baseline: 24.7177x; 1.5286x over previous
import functools

import jax
import jax.numpy as jnp
from jax import lax
from jax.experimental import pallas as pl
from jax.experimental.pallas import tpu as pltpu
from jax.experimental.pallas import tpu_sc as plsc

B = 64
S = 30
H = 20
MAXV = 256
PAD = 512
N = B * S

PER_W = 128
ACT_W = N // PER_W
HALF = PER_W // 2

_LN2 = 0.6931471805599453
_SQRT2 = 1.4142135381698608


def _vlog(x):
    xi = lax.bitcast_convert_type(x, jnp.int32)
    e = lax.shift_right_arithmetic(xi, 23) - 127
    m = lax.bitcast_convert_type((xi & 0x007FFFFF) | 0x3F800000, jnp.float32)
    big = m > _SQRT2
    e = e + big.astype(jnp.int32)
    m = jnp.where(big, m * 0.5, m)
    t = (m - 1.0) / (m + 1.0)
    z = t * t
    p = 2.0 + z * (0.66666668653488159 + z * (0.40000000596046448
        + z * (0.28571429848670959 + z * 0.22222222222222222)))
    return e.astype(jnp.float32) * _LN2 + t * p


def _tc_prep(score_ref, upd_ref, cm_ref, ct_ref, ns_ref, ne_ref, nm_ref,
             wcls_ref, wext_ref, cto_ref, nso_ref, neo_ref, r_ref):
    best = score_ref[0]
    bidx = jnp.zeros((S, B), jnp.int32)
    for h in range(1, H):
        v = score_ref[h]
        better = v > best
        best = jnp.where(better, v, best)
        bidx = jnp.where(better, h, bidx)

    def pick(ref):
        acc = ref[0]
        for h in range(1, H):
            acc = jnp.where(bidx == h, ref[h], acc)
        return acc

    ok = upd_ref[...] == 1
    wcls = ((pick(cm_ref) == 1) & ok).astype(jnp.int32)
    wext = ((pick(nm_ref) == 1) & ok).astype(jnp.int32)
    s_iota = lax.broadcasted_iota(jnp.int32, (S, B), 0)
    b_iota = lax.broadcasted_iota(jnp.int32, (S, B), 1)
    r = s_iota * N + b_iota * S + s_iota
    rows = pl.ds(0, S)
    wcls_ref[rows, :] = wcls
    wext_ref[rows, :] = wext
    cto_ref[rows, :] = pick(ct_ref)
    nso_ref[rows, :] = pick(ns_ref)
    neo_ref[rows, :] = pick(ne_ref)
    r_ref[rows, :] = r


_I32SB = jax.ShapeDtypeStruct((32, B), jnp.int32)


@functools.partial(
    pl.kernel,
    mesh=plsc.VectorSubcoreMesh(core_axis_name="c", subcore_axis_name="s",
                                num_cores=1),
    out_type=jax.ShapeDtypeStruct((16,), jnp.float32),
    compiler_params=pltpu.CompilerParams(needs_layout_passes=False,
                                         disable_bounds_checks=True),
    scratch_types=[
        pltpu.VMEM((8, B), jnp.int32),
        pltpu.VMEM((8, B), jnp.int32),
        pltpu.VMEM((8, B), jnp.int32),
        pltpu.VMEM((8, B), jnp.int32),
        pltpu.VMEM((8, B), jnp.int32),
        pltpu.VMEM((8, B), jnp.int32),
        pltpu.VMEM((HALF, MAXV), jnp.float32),
        pltpu.VMEM((HALF, PAD), jnp.float32),
        pltpu.VMEM((HALF, PAD), jnp.float32),
        pltpu.VMEM((5 * 16,), jnp.float32),
        pltpu.VMEM((ACT_W * 5 * 16,), jnp.float32),
        pltpu.VMEM((16,), jnp.float32),
        pltpu.VMEM((16,), jnp.float32),
        pltpu.VMEM_SHARED((ACT_W * 5 * 16,), jnp.float32),
        pltpu.SemaphoreType.DMA,
    ],
)
def _sc_loss(wcls_hbm, wext_hbm, ct_hbm, ns_hbm, ne_hbm, r_hbm,
             svp_hbm, sp_hbm, ep_hbm, out_hbm,
             wcls_v, wext_v, ct_v, ns_v, ne_v, r_v,
             svpv_v, spv_v, epv_v,
             acc_v, all_v, tmp_v, outv_v, shared, sem):
    wid = lax.axis_index("s")

    @pl.when(wid < ACT_W)
    def _work():
        sb = pl.multiple_of((2 * wid) & ~7, 8)
        o = (2 * wid) & 7
        blk = pl.ds(sb, 8)
        cps = [pltpu.async_copy(wcls_hbm.at[blk], wcls_v, sem),
               pltpu.async_copy(wext_hbm.at[blk], wext_v, sem),
               pltpu.async_copy(ct_hbm.at[blk], ct_v, sem),
               pltpu.async_copy(ns_hbm.at[blk], ns_v, sem),
               pltpu.async_copy(ne_hbm.at[blk], ne_v, sem),
               pltpu.async_copy(r_hbm.at[blk], r_v, sem)]
        for cp in cps:
            cp.wait()
        fl_svp = svp_hbm.reshape(S * N, MAXV)
        fl_sp = sp_hbm.reshape(S * N, PAD)
        fl_ep = ep_hbm.reshape(S * N, PAD)
        lanes = lax.broadcasted_iota(jnp.int32, (16,), 0)
        zero = jnp.zeros((16,), jnp.float32)
        cls_sum, cls_cnt = zero, zero
        s_sum, e_sum, ext_cnt = zero, zero, zero
        for half in range(2):
            idx_ref = r_v.at[o + half]
            cps = [pltpu.async_copy(fl_svp.at[idx_ref], svpv_v, sem),
                   pltpu.async_copy(fl_sp.at[idx_ref], spv_v, sem),
                   pltpu.async_copy(fl_ep.at[idx_ref], epv_v, sem)]
            for cp in cps:
                cp.wait()
            row_b = jnp.full((16,), o + half, jnp.int32)
            for c in range(HALF // 16):
                rows = c * 16 + lanes
                col = c * 16 + lanes
                ctv = plsc.load_gather(ct_v, [row_b, col])
                nsv = plsc.load_gather(ns_v, [row_b, col])
                nev = plsc.load_gather(ne_v, [row_b, col])
                sv = plsc.load_gather(svpv_v, [rows, ctv])
                st = plsc.load_gather(spv_v, [rows, nsv])
                en = plsc.load_gather(epv_v, [rows, nev])
                wcls = plsc.load_gather(wcls_v, [row_b, col]) == 1
                wext = plsc.load_gather(wext_v, [row_b, col]) == 1
                cls_sum = cls_sum + jnp.where(wcls, -_vlog(sv + 1e-5), 0.0)
                cls_cnt = cls_cnt + jnp.where(wcls, 1.0, 0.0)
                s_sum = s_sum + jnp.where(wext, -_vlog(st + 1e-5), 0.0)
                e_sum = e_sum + jnp.where(wext, -_vlog(en + 1e-5), 0.0)
                ext_cnt = ext_cnt + jnp.where(wext, 1.0, 0.0)
        acc_v[pl.ds(0, 16)] = cls_sum
        acc_v[pl.ds(16, 16)] = cls_cnt
        acc_v[pl.ds(32, 16)] = s_sum
        acc_v[pl.ds(48, 16)] = e_sum
        acc_v[pl.ds(64, 16)] = ext_cnt
        pltpu.sync_copy(acc_v, shared.at[pl.ds(wid * 80, 80)])

    plsc.subcore_barrier()

    @pl.when(wid == 0)
    def _reduce():
        pltpu.sync_copy(shared, all_v)
        tot = [jnp.zeros((16,), jnp.float32) for _ in range(5)]
        for w in range(ACT_W):
            for rr in range(5):
                tot[rr] = tot[rr] + all_v[pl.ds(w * 80 + rr * 16, 16)]
        full15 = jnp.full((16,), 15, jnp.int32)

        def lanesum(v):
            tmp_v[...] = jnp.cumsum(v)
            return plsc.load_gather(tmp_v, [full15])

        cls_sum = lanesum(tot[0])
        cls_cnt = lanesum(tot[1])
        s_sum = lanesum(tot[2])
        e_sum = lanesum(tot[3])
        ext_cnt = lanesum(tot[4])
        cls = jnp.where(cls_cnt > 0, cls_sum / jnp.maximum(cls_cnt, 1.0), 0.0)
        stl = jnp.where(ext_cnt > 0, s_sum / jnp.maximum(ext_cnt, 1.0), 0.0)
        enl = jnp.where(ext_cnt > 0, e_sum / jnp.maximum(ext_cnt, 1.0), 0.0)
        outv_v[...] = cls + stl + enl
        pltpu.sync_copy(outv_v, out_hbm)


def kernel(score, update_slot, startProb, endProb, slotValueProb,
           cata_target, cate_mask, noncate_start, noncate_end, noncate_mask):
    tr = lambda t: t.astype(jnp.int32).transpose(1, 2, 0)
    wcls, wext, ct, ns, ne, r = pl.pallas_call(
        _tc_prep,
        out_shape=[_I32SB] * 6,
    )(
        score.transpose(2, 1, 0),
        update_slot.astype(jnp.int32).T,
        tr(cate_mask),
        tr(cata_target),
        tr(noncate_start),
        tr(noncate_end),
        tr(noncate_mask),
    )
    out16 = _sc_loss(
        wcls, wext, ct, ns, ne, r,
        slotValueProb.transpose(1, 0, 2),
        startProb.transpose(1, 0, 2),
        endProb.transpose(1, 0, 2),
    )
    return out16[0]
